# baseline (speedup 1.0000x reference)
import functools

import jax
import jax.numpy as jnp
from jax import lax
from jax.experimental import pallas as pl
from jax.experimental.pallas import tpu as pltpu
from jax.experimental.pallas import tpu_sc as plsc

_VOCAB = 1000
_DMODEL = 128
_BATCH = 4096
_SEQ = 20

_NTOK = _BATCH * _SEQ
_NW = 32
_BPW = _NTOK // _NW
_CH = 64
_NCH = _BPW // _CH


def _matmul_body(emb_ref, w_ref, m_ref):
    m_ref[...] = lax.dot_general(
        emb_ref[...], w_ref[...],
        dimension_numbers=(((1,), (1,)), ((), ())),
        preferred_element_type=jnp.float32,
    )


def _fused_table(embedding, lm_head_w):
    return pl.pallas_call(
        _matmul_body,
        out_shape=jax.ShapeDtypeStruct((_VOCAB, _VOCAB), jnp.float32),
    )(embedding, lm_head_w)


def _gather_body(table_hbm, idx_hbm, out_hbm, idx_v, rows_v, tab_sh, gsem):
    cid = lax.axis_index("c")
    sid = lax.axis_index("s")
    wid = sid * 2 + cid

    @pl.when(sid == 0)
    def _():
        pltpu.sync_copy(table_hbm, tab_sh)

    pltpu.sync_copy(idx_hbm.at[wid], idx_v)
    plsc.subcore_barrier()

    base = wid * _BPW

    def chunk(j, carry):
        pltpu.async_copy(tab_sh.at[idx_v.at[j]], rows_v, gsem).wait()
        pltpu.sync_copy(rows_v, out_hbm.at[pl.ds(base + j * _CH, _CH)])
        return carry

    lax.fori_loop(0, _NCH, chunk, 0, unroll=False)


@functools.partial(
    pl.kernel,
    out_type=jax.ShapeDtypeStruct((_NTOK, _VOCAB), jnp.float32),
    mesh=plsc.VectorSubcoreMesh(core_axis_name="c", subcore_axis_name="s"),
    scratch_types=[
        pltpu.VMEM((_NCH, _CH), jnp.int32),
        pltpu.VMEM((_CH, _VOCAB), jnp.float32),
        pltpu.VMEM_SHARED((_VOCAB, _VOCAB), jnp.float32),
        pltpu.SemaphoreType.DMA,
    ],
    compiler_params=pltpu.CompilerParams(use_tc_tiling_on_sc=False),
)
def _gather_call(table_hbm, idx_hbm, out_hbm, idx_v, rows_v, tab_sh, gsem):
    _gather_body(table_hbm, idx_hbm, out_hbm, idx_v, rows_v, tab_sh, gsem)


def kernel(input_ids, embedding, lm_head_w):
    m = _fused_table(embedding, lm_head_w)
    ids = input_ids.astype(jnp.int32).reshape(_NW, _NCH, _CH)
    out = _gather_call(m, ids)
    return out.reshape(_BATCH, _SEQ, _VOCAB)


# speedup vs baseline: 1.0956x; 1.0956x over previous
import functools

import jax
import jax.numpy as jnp
from jax import lax
from jax.experimental import pallas as pl
from jax.experimental.pallas import tpu as pltpu
from jax.experimental.pallas import tpu_sc as plsc

_VOCAB = 1000
_DMODEL = 128
_BATCH = 4096
_SEQ = 20

_NTOK = _BATCH * _SEQ
_NW = 32
_BPW = _NTOK // _NW
_CH = 16
_NCH = _BPW // _CH
_NBUF = 4


def _matmul_body(emb_ref, w_ref, m_ref):
    m_ref[...] = lax.dot_general(
        emb_ref[...], w_ref[...],
        dimension_numbers=(((1,), (1,)), ((), ())),
        preferred_element_type=jnp.float32,
    )


def _fused_table(embedding, lm_head_w):
    return pl.pallas_call(
        _matmul_body,
        out_shape=jax.ShapeDtypeStruct((_VOCAB, _VOCAB), jnp.float32),
    )(embedding, lm_head_w)


def _gather_body(table_hbm, idx_hbm, out_hbm, idx_v, rows_v, tab_sh, gsem, ssem):
    cid = lax.axis_index("c")
    sid = lax.axis_index("s")
    wid = sid * 2 + cid

    @pl.when(sid == 0)
    def _():
        pltpu.sync_copy(table_hbm, tab_sh)

    pltpu.sync_copy(idx_hbm.at[wid], idx_v)
    plsc.subcore_barrier()

    base = wid * _BPW

    def start_gather(j, b):
        pltpu.async_copy(tab_sh.at[idx_v.at[j]], rows_v.at[b], gsem.at[b])

    def wait_gather(j, b):
        pltpu.make_async_copy(
            tab_sh.at[idx_v.at[j]], rows_v.at[b], gsem.at[b]).wait()

    def out_slice(j):
        return out_hbm.at[pl.ds(base + j * _CH, _CH)]

    def start_scatter(j, b):
        pltpu.async_copy(rows_v.at[b], out_slice(j), ssem.at[b])

    def wait_scatter(j, b):
        pltpu.make_async_copy(rows_v.at[b], out_slice(j), ssem.at[b]).wait()

    def step(j, b):
        wait_gather(j, b)
        start_scatter(j, b)
        b2 = (b + 2) % _NBUF
        wait_scatter(j - 2, b2)
        start_gather(j + 2, b2)

    start_gather(0, 0)
    start_gather(1, 1)
    for j in (0, 1):
        wait_gather(j, j)
        start_scatter(j, j)
        start_gather(j + 2, j + 2)
    for j in (2, 3):
        step(j, j)

    def block(jj, carry):
        j0 = jj * _NBUF
        for b in range(_NBUF):
            step(j0 + b, b)
        return carry

    lax.fori_loop(1, _NCH // _NBUF - 1, block, 0, unroll=False)

    for b in (0, 1):
        step(_NCH - 4 + b, b)
    for b in (2, 3):
        j = _NCH - 4 + b
        wait_gather(j, b)
        start_scatter(j, b)
    for b in range(_NBUF):
        wait_scatter(_NCH - 4 + b, b)


@functools.partial(
    pl.kernel,
    out_type=jax.ShapeDtypeStruct((_NTOK, _VOCAB), jnp.float32),
    mesh=plsc.VectorSubcoreMesh(core_axis_name="c", subcore_axis_name="s"),
    scratch_types=[
        pltpu.VMEM((_NCH, _CH), jnp.int32),
        pltpu.VMEM((_NBUF, _CH, _VOCAB), jnp.float32),
        pltpu.VMEM_SHARED((_VOCAB, _VOCAB), jnp.float32),
        pltpu.SemaphoreType.DMA((_NBUF,)),
        pltpu.SemaphoreType.DMA((_NBUF,)),
    ],
    compiler_params=pltpu.CompilerParams(use_tc_tiling_on_sc=False),
)
def _gather_call(table_hbm, idx_hbm, out_hbm, idx_v, rows_v, tab_sh, gsem, ssem):
    _gather_body(table_hbm, idx_hbm, out_hbm, idx_v, rows_v, tab_sh, gsem, ssem)


def kernel(input_ids, embedding, lm_head_w):
    m = _fused_table(embedding, lm_head_w)
    ids = input_ids.astype(jnp.int32).reshape(_NW, _NCH, _CH)
    out = _gather_call(m, ids)
    return out.reshape(_BATCH, _SEQ, _VOCAB)
